# Initial kernel scaffold; baseline (speedup 1.0000x reference)
import jax
import jax.numpy as jnp
from jax.experimental import pallas as pl


def kernel(in_feat, edge_index, labels, train_mask, W1, b1, W2, b2):
    raise NotImplementedError("write your pallas kernel here")


# speedup vs baseline: 1.3457x; 1.3457x over previous
import jax, jax.numpy as jnp
import numpy as np

N, E, D, H, C = 10000, 320000, 128, 256, 40


def kernel(in_feat, edge_index, labels, train_mask, W1, b1, W2, b2):
    src, dst = edge_index[0], edge_index[1]
    X = in_feat
    P1 = X @ W1

    deg0 = jnp.zeros((N,), jnp.float32).at[dst].add(1.0) + 1.0
    dinv0 = 1.0 / jnp.sqrt(deg0)
    norm0 = dinv0[src] * dinv0[dst]
    sl0 = dinv0 * dinv0

    Z1 = jnp.zeros((N, H), jnp.float32).at[dst].add(norm0[:, None] * P1[src])
    Z1 = Z1 + sl0[:, None] * P1 + b1
    H1 = jnp.maximum(Z1, 0.0)
    P2 = H1 @ W2
    logits = jnp.zeros((N, C), jnp.float32).at[dst].add(norm0[:, None] * P2[src])
    logits = logits + sl0[:, None] * P2 + b2

    m = jnp.max(logits, axis=-1, keepdims=True)
    p = jnp.exp(logits - m)
    p = p / jnp.sum(p, axis=-1, keepdims=True)
    G = (p - jax.nn.one_hot(labels, C, dtype=jnp.float32)) / N

    dP2 = jnp.zeros((N, C), jnp.float32).at[src].add(norm0[:, None] * G[dst])
    dP2 = dP2 + sl0[:, None] * G
    dnorm2 = jnp.sum(G[dst] * P2[src], axis=-1)
    dnorm_sl2 = jnp.sum(G * P2, axis=-1)
    dH1 = dP2 @ W2.T
    dZ1 = dH1 * (Z1 > 0)
    dnorm1 = jnp.sum(dZ1[dst] * P1[src], axis=-1)
    dnorm_sl1 = jnp.sum(dZ1 * P1, axis=-1)
    dnorm = dnorm1 + dnorm2
    dnorm_sl = dnorm_sl1 + dnorm_sl2

    ddinv = jnp.zeros((N,), jnp.float32).at[src].add(dnorm * dinv0[dst])
    ddinv = ddinv.at[dst].add(dnorm * dinv0[src])
    ddinv = ddinv + 2.0 * dinv0 * dnorm_sl
    ddeg = -0.5 * ddinv * dinv0 ** 3
    g = dinv0[src] * dinv0[dst] * dnorm + ddeg[dst] - 1.0 / jnp.sqrt(jnp.float32(E))

    ew = jnp.clip(1.0 - 0.01 * g / (jnp.abs(g) + 1e-8), 0.0, 1.0)

    deg2 = jnp.zeros((N,), jnp.float32).at[dst].add(ew) + 1.0
    dinv2 = 1.0 / jnp.sqrt(deg2)
    norm2 = dinv2[src] * ew * dinv2[dst]
    sl2 = dinv2 * dinv2
    Z1b = jnp.zeros((N, H), jnp.float32).at[dst].add(norm2[:, None] * P1[src])
    Z1b = Z1b + sl2[:, None] * P1 + b1
    H1b = jnp.maximum(Z1b, 0.0)
    P2b = H1b @ W2
    out = jnp.zeros((N, C), jnp.float32).at[dst].add(norm2[:, None] * P2b[src])
    out = out + sl2[:, None] * P2b + b2
    return out


# speedup vs baseline: 1.7184x; 1.2769x over previous
import functools

import jax
import jax.numpy as jnp
from jax import lax
from jax.experimental import pallas as pl
from jax.experimental.pallas import tpu as pltpu
from jax.experimental.pallas import tpu_sc as plsc

N = 10000
E = 320000
D = 128
H = 256
C = 40
CP = 48
TW = 384

NC, NS, LN = 2, 16, 16
NW = NC * NS
EPT = E // NW

BR = 2000
GRID = N // BR
BE = 4000
EGRID = E // BE
BP = 256
PGRID = E // BP

NB = 10240
BPT = NB // NW

WN = 1000
NWIN = 10
WR = 1008

f32 = jnp.float32
i32 = jnp.int32

_MESH = plsc.VectorSubcoreMesh(core_axis_name="c", subcore_axis_name="s",
                               num_cores=NC, num_subcores=NS)


def _tc_call(body, out_shapes, in_specs, out_specs, args, grid=(GRID,),
             scratch=()):
    return pl.pallas_call(
        body,
        grid=grid,
        in_specs=in_specs,
        out_specs=out_specs,
        out_shape=out_shapes,
        scratch_shapes=list(scratch),
    )(*args)


def _rb(*dims, br=BR):
    def im(i):
        return tuple(i if d == -1 else 0 for d in dims)
    shape = tuple(br if d == -1 else d for d in dims)
    return pl.BlockSpec(shape, im)


def _full(*shape):
    return pl.BlockSpec(shape, lambda i: tuple(0 for _ in shape))


def _wspec():
    return pl.BlockSpec((2, 2, WR, 16), lambda i: (0, i, 0, 0))


def tc_mm1(x, w1):
    def body(x_ref, w_ref, o_ref):
        o_ref[...] = jnp.dot(x_ref[...], w_ref[...],
                             preferred_element_type=f32)
    return _tc_call(body, jax.ShapeDtypeStruct((N, H), f32),
                    [_rb(-1, D), _full(D, H)], _rb(-1, H), (x, w1))


def tc_winred(parts, self_w, emit_dinv):
    def body(p_ref, c_ref, *rest):
        cnt = jnp.concatenate(
            [p_ref[0, 0, :WN, 0] + p_ref[1, 0, :WN, 0],
             p_ref[0, 1, :WN, 0] + p_ref[1, 1, :WN, 0]])
        c_ref[...] = cnt[:, None]
        if emit_dinv:
            rest[0][...] = lax.rsqrt(cnt + self_w)[:, None]
    shapes = [jax.ShapeDtypeStruct((N, 1), f32)]
    ospecs = [_rb(-1, 1)]
    if emit_dinv:
        shapes.append(jax.ShapeDtypeStruct((N, 1), f32))
        ospecs.append(_rb(-1, 1))
    return _tc_call(body, tuple(shapes), [_wspec()], tuple(ospecs), (parts,))


def tc_ends(counts2):
    def body(c_ref, a_ref, b_ref, am_ref, bm_ref):
        x = c_ref[...]
        u125 = (lax.broadcasted_iota(i32, (125, 125), 0)
                <= lax.broadcasted_iota(i32, (125, 125), 1)).astype(f32)
        l80s = (lax.broadcasted_iota(i32, (80, 80), 0)
                > lax.broadcasted_iota(i32, (80, 80), 1)).astype(f32)
        within = jnp.dot(x, u125, preferred_element_type=f32)
        rowtot = jnp.sum(x, axis=1)
        prev = jnp.dot(l80s, rowtot[:, None], preferred_element_type=f32)
        ends = within + prev
        a_raw = ends - 1.0
        b_raw = ends - x - 1.0
        am = (x > 0.0).astype(f32)
        bm = am * (b_raw >= 0.0).astype(f32)
        a_ref[...] = jnp.maximum(a_raw, 0.0).astype(i32)
        b_ref[...] = jnp.maximum(b_raw, 0.0).astype(i32)
        am_ref[...] = am
        bm_ref[...] = bm
    fullspec = pl.BlockSpec((80, 125), lambda: (0, 0))
    return _tc_call(
        body,
        (jax.ShapeDtypeStruct((80, 125), i32),
         jax.ShapeDtypeStruct((80, 125), i32),
         jax.ShapeDtypeStruct((80, 125), f32),
         jax.ShapeDtypeStruct((80, 125), f32)),
        [fullspec], (fullspec, fullspec, fullspec, fullspec),
        (counts2,), grid=())


def tc_scale(dinv, p1):
    def body(d_ref, p_ref, o_ref):
        o_ref[...] = d_ref[:, 0][:, None] * p_ref[...]
    return _tc_call(body, jax.ShapeDtypeStruct((N, H), f32),
                    [_rb(-1, 1), _rb(-1, H)], _rb(-1, H), (dinv, p1))


def tc_prefix(rows, ltri, w, width):
    weighted = w is not None

    def body(*refs):
        if weighted:
            r_ref, l_ref, w_ref, o_ref, carry = refs
        else:
            r_ref, l_ref, o_ref, carry = refs

        @pl.when(pl.program_id(0) == 0)
        def _():
            carry[...] = jnp.zeros((1, width), f32)
        x = r_ref[...]
        if weighted:
            x = x * w_ref[...]
        p = jnp.dot(l_ref[...], x, preferred_element_type=f32) + carry[...]
        o_ref[...] = p
        carry[...] = p[BP - 1:BP, :]

    in_specs = [_rb(-1, width, br=BP), _full(BP, BP)]
    args = [rows, ltri]
    if weighted:
        in_specs.append(_rb(-1, 1, br=BP))
        args.append(w)
    return _tc_call(body, jax.ShapeDtypeStruct((E, width), f32),
                    in_specs, _rb(-1, width, br=BP), tuple(args),
                    grid=(PGRID,),
                    scratch=[pltpu.VMEM((1, width), f32)])


def tc_prefix1(rows16, ltri):
    def body(r_ref, l_ref, o_ref, carry):
        @pl.when(pl.program_id(0) == 0)
        def _():
            carry[...] = jnp.zeros((1, 128), f32)
        p = jnp.dot(l_ref[...], r_ref[:, 0:1],
                    preferred_element_type=f32) + carry[:, 0:1]
        o_ref[...] = jnp.broadcast_to(p, (BP, 128))
        carry[...] = jnp.broadcast_to(p[BP - 1:BP, :], (1, 128))
    return _tc_call(body, jax.ShapeDtypeStruct((E, 128), f32),
                    [_rb(-1, 16, br=BP), _full(BP, BP)],
                    _rb(-1, 128, br=BP), (rows16, ltri),
                    grid=(PGRID,),
                    scratch=[pltpu.VMEM((1, 128), f32)])


def tc_z1(pa, pb, am, bm, p1, dinv, b1, w2p, emit_h):
    def body(pa_ref, pb_ref, am_ref, bm_ref, p_ref, d_ref, b_ref, w_ref,
             *outs):
        dinv = d_ref[:, 0]
        agg = am_ref[...] * pa_ref[...] - bm_ref[...] * pb_ref[...]
        z = dinv[:, None] * agg + (dinv * dinv)[:, None] * p_ref[...] \
            + b_ref[...]
        h = jnp.maximum(z, 0.0)
        p2 = jnp.dot(h, w_ref[...], preferred_element_type=f32)
        if emit_h:
            outs[0][...] = h
            outs = outs[1:]
        outs[0][...] = p2
        outs[1][...] = jnp.concatenate(
            [dinv[:, None] * p2, jnp.zeros((BR, 128 - CP), f32)], axis=-1)
    shapes = ([jax.ShapeDtypeStruct((N, H), f32)] if emit_h else []) + [
        jax.ShapeDtypeStruct((N, CP), f32), jax.ShapeDtypeStruct((N, 128), f32)]
    ospecs = ([_rb(-1, H)] if emit_h else []) + [_rb(-1, CP), _rb(-1, 128)]
    return _tc_call(
        body, tuple(shapes),
        [_rb(-1, H), _rb(-1, H), _rb(-1, 1), _rb(-1, 1), _rb(-1, H),
         _rb(-1, 1), _full(1, H), _full(H, CP)],
        tuple(ospecs),
        (pa, pb, am, bm, p1, dinv, b1.reshape(1, H), w2p))


def tc_softmax(pa, pb, am, bm, p2p, dinv, b2p, labels):
    def body(pa_ref, pb_ref, am_ref, bm_ref, p2_ref, d_ref, b_ref, l_ref,
             g_ref, gs_ref):
        dinv = d_ref[:, 0]
        agg = am_ref[...] * pa_ref[:, :CP] - bm_ref[...] * pb_ref[:, :CP]
        logits = (dinv[:, None] * agg
                  + (dinv * dinv)[:, None] * p2_ref[...] + b_ref[...])
        colm = lax.broadcasted_iota(i32, (BR, CP), 1) < C
        lm = jnp.max(jnp.where(colm, logits, -1e30), axis=-1, keepdims=True)
        ex = jnp.where(colm, jnp.exp(logits - lm), 0.0)
        sm = ex / jnp.sum(ex, axis=-1, keepdims=True)
        onehot = (lax.broadcasted_iota(i32, (BR, CP), 1)
                  == l_ref[:, 0][:, None]).astype(f32)
        g = jnp.where(colm, (sm - onehot) * (1.0 / N), 0.0)
        g_ref[...] = g
        gs_ref[...] = jnp.concatenate(
            [dinv[:, None] * g, jnp.zeros((BR, 128 - CP), f32)], axis=-1)
    return _tc_call(
        body,
        (jax.ShapeDtypeStruct((N, CP), f32), jax.ShapeDtypeStruct((N, 128), f32)),
        [_rb(-1, 128), _rb(-1, 128), _rb(-1, 1), _rb(-1, 1), _rb(-1, CP),
         _rb(-1, 1), _full(1, CP), _rb(-1, 1)],
        (_rb(-1, CP), _rb(-1, 128)),
        (pa, pb, am, bm, p2p, dinv, b2p.reshape(1, CP), labels))


def tc_dz1(pa, pb, am, bm, gp, p1, h1, dinv, w2pt, p2p):
    def body(pa_ref, pb_ref, am_ref, bm_ref, g_ref, p1_ref, h_ref, d_ref,
             w_ref, p2_ref, td_ref, ts_ref, sl_ref):
        dinv = d_ref[:, 0]
        agg = am_ref[...] * pa_ref[:, :CP] - bm_ref[...] * pb_ref[:, :CP]
        dp2 = dinv[:, None] * agg + (dinv * dinv)[:, None] * g_ref[...]
        dh1 = jnp.dot(dp2, w_ref[...], preferred_element_type=f32)
        dz1 = jnp.where(h_ref[...] > 0.0, dh1, 0.0)
        zpad = jnp.zeros((BR, TW - H - CP), f32)
        td_ref[...] = jnp.concatenate(
            [dinv[:, None] * dz1, dinv[:, None] * g_ref[...], zpad], axis=-1)
        ts_ref[...] = jnp.concatenate(
            [dinv[:, None] * p1_ref[...], dinv[:, None] * p2_ref[...], zpad],
            axis=-1)
        sl_ref[...] = (jnp.sum(g_ref[...] * p2_ref[...], axis=-1)
                       + jnp.sum(dz1 * p1_ref[...], axis=-1))[:, None]
    return _tc_call(
        body,
        (jax.ShapeDtypeStruct((N, TW), f32), jax.ShapeDtypeStruct((N, TW), f32),
         jax.ShapeDtypeStruct((N, 1), f32)),
        [_rb(-1, 128), _rb(-1, 128), _rb(-1, 1), _rb(-1, 1), _rb(-1, CP),
         _rb(-1, H), _rb(-1, H), _rb(-1, 1), _full(CP, H), _rb(-1, CP)],
        (_rb(-1, TW), _rb(-1, TW), _rb(-1, 1)),
        (pa, pb, am, bm, gp, p1, h1, dinv, w2pt, p2p))


def tc_dots(ud, vs):
    def body(u_ref, v_ref, o_ref):
        t1 = jnp.sum(u_ref[...] * v_ref[...], axis=-1)
        o_ref[...] = jnp.broadcast_to(t1[:, None], (BE, 16))
    return _tc_call(
        body, jax.ShapeDtypeStruct((E, 16), f32),
        [_rb(-1, TW, br=BE), _rb(-1, TW, br=BE)],
        _rb(-1, 16, br=BE), (ud, vs), grid=(EGRID,))


def tc_ddeg(pas, pbs, pad_, pbd, ams, bms, amd, bmd, dinv, dnsl):
    def body(pas_ref, pbs_ref, pad_ref, pbd_ref, ams_ref, bms_ref,
             amd_ref, bmd_ref, d_ref, s_ref, o_ref):
        dinv = d_ref[:, 0]
        t1s = ams_ref[:, 0] * pas_ref[:, 0] - bms_ref[:, 0] * pbs_ref[:, 0]
        t1d = amd_ref[:, 0] * pad_ref[:, 0] - bmd_ref[:, 0] * pbd_ref[:, 0]
        ddinv = (t1s + t1d) / dinv + 2.0 * dinv * s_ref[:, 0]
        ddeg = -0.5 * ddinv * dinv * dinv * dinv
        o_ref[...] = jnp.broadcast_to(ddeg[:, None], (BR, 128))
    return _tc_call(
        body, jax.ShapeDtypeStruct((N, 128), f32),
        [_rb(-1, 128), _rb(-1, 128), _rb(-1, 128), _rb(-1, 128),
         _rb(-1, 1), _rb(-1, 1), _rb(-1, 1), _rb(-1, 1),
         _rb(-1, 1), _rb(-1, 1)],
        _rb(-1, 128), (pas, pbs, pad_, pbd, ams, bms, amd, bmd, dinv, dnsl))


def tc_edgeup(t1rows, dd):
    CE = 1.0 / float(E) ** 0.5

    def body(t_ref, d_ref, ew_ref, er_ref):
        g = t_ref[:, 0] + d_ref[:, 0] - CE
        ew = 1.0 - 0.01 * g / (jnp.abs(g) + 1e-8)
        ew = jnp.clip(ew, 0.0, 1.0)
        ew_ref[...] = ew[:, None]
        er_ref[...] = jnp.broadcast_to(ew[:, None], (BE, 16))
    return _tc_call(
        body,
        (jax.ShapeDtypeStruct((E, 1), f32), jax.ShapeDtypeStruct((E, 16), f32)),
        [_rb(-1, 16, br=BE), _rb(-1, 128, br=BE)],
        (_rb(-1, 1, br=BE), _rb(-1, 16, br=BE)),
        (t1rows, dd), grid=(EGRID,))


def tc_deg2scale(pa, pb, am, bm, p1):
    def body(pa_ref, pb_ref, am_ref, bm_ref, p_ref, d_ref, o_ref):
        ewsum = am_ref[:, 0] * pa_ref[:, 0] - bm_ref[:, 0] * pb_ref[:, 0]
        dinv2 = lax.rsqrt(ewsum + 1.0)
        d_ref[...] = dinv2[:, None]
        o_ref[...] = dinv2[:, None] * p_ref[...]
    return _tc_call(
        body,
        (jax.ShapeDtypeStruct((N, 1), f32), jax.ShapeDtypeStruct((N, H), f32)),
        [_rb(-1, 128), _rb(-1, 128), _rb(-1, 1), _rb(-1, 1), _rb(-1, H)],
        (_rb(-1, 1), _rb(-1, H)), (pa, pb, am, bm, p1))


def tc_final(pa, pb, am, bm, p2bp, dinv2, b2):
    def body(pa_ref, pb_ref, am_ref, bm_ref, p2_ref, d_ref, b_ref, o_ref):
        dinv = d_ref[:, 0]
        agg = am_ref[...] * pa_ref[:, :C] - bm_ref[...] * pb_ref[:, :C]
        o_ref[...] = (dinv[:, None] * agg
                      + (dinv * dinv)[:, None] * p2_ref[:, :C] + b_ref[...])
    return _tc_call(
        body, jax.ShapeDtypeStruct((N, C), f32),
        [_rb(-1, 128), _rb(-1, 128), _rb(-1, 1), _rb(-1, 1), _rb(-1, CP),
         _rb(-1, 1), _full(1, C)],
        _rb(-1, C), (pa, pb, am, bm, p2bp, dinv2, b2.reshape(1, C)))


def sc_gather(table, idx, width, mtot, g):
    per_tile = mtot // NW

    @functools.partial(
        pl.kernel,
        out_type=jax.ShapeDtypeStruct((mtot, width), f32),
        mesh=_MESH,
        scratch_types=[
            pltpu.VMEM((g,), i32), pltpu.VMEM((g, width), f32),
            pltpu.SemaphoreType.DMA,
        ])
    def run(tab_h, gi_h, out_h, gi_v, rows_v, sem):
        wid = lax.axis_index("s") * NC + lax.axis_index("c")

        def chunk(k, _):
            base = wid * per_tile + k * g
            pltpu.sync_copy(gi_h.at[pl.ds(base, g)], gi_v)
            pltpu.async_copy(tab_h.at[gi_v], rows_v, sem).wait()
            pltpu.sync_copy(rows_v, out_h.at[pl.ds(base, g)])
            return 0
        lax.fori_loop(0, per_tile // g, chunk, 0)

    return run(table, idx)


def sc_gather2(tu, tv, uidx, vidx, width, mtot, g):
    per_tile = mtot // NW

    @functools.partial(
        pl.kernel,
        out_type=(jax.ShapeDtypeStruct((mtot, width), f32),
                  jax.ShapeDtypeStruct((mtot, width), f32)),
        mesh=_MESH,
        scratch_types=[
            pltpu.VMEM((g,), i32), pltpu.VMEM((g,), i32),
            pltpu.VMEM((g, width), f32), pltpu.VMEM((g, width), f32),
            pltpu.SemaphoreType.DMA, pltpu.SemaphoreType.DMA,
        ])
    def run(tu_h, tv_h, ui_h, vi_h, ou_h, ov_h,
            ui_v, vi_v, u_v, v_v, sem1, sem2):
        wid = lax.axis_index("s") * NC + lax.axis_index("c")

        def chunk(k, _):
            base = wid * per_tile + k * g
            pltpu.sync_copy(ui_h.at[pl.ds(base, g)], ui_v)
            pltpu.sync_copy(vi_h.at[pl.ds(base, g)], vi_v)
            cp1 = pltpu.async_copy(tu_h.at[ui_v], u_v, sem1)
            cp2 = pltpu.async_copy(tv_h.at[vi_v], v_v, sem2)
            cp1.wait()
            cp2.wait()
            pltpu.sync_copy(u_v, ou_h.at[pl.ds(base, g)])
            pltpu.sync_copy(v_v, ov_h.at[pl.ds(base, g)])
            return 0
        lax.fori_loop(0, per_tile // g, chunk, 0)

    return run(tu, tv, uidx, vidx)


def sc_win16(idx, ones_g, zeros_w):
    G = 400
    NSUB, SUB = 5, 80

    @functools.partial(
        pl.kernel,
        out_type=jax.ShapeDtypeStruct((NC, NWIN, WR, 16), f32),
        mesh=_MESH,
        scratch_types=[
            pltpu.VMEM((G,), i32),
            pltpu.VMEM((NSUB, SUB), i32),
            pltpu.VMEM((G, 16), f32),
            pltpu.VMEM_SHARED((WR, 16), f32),
        ])
    def run(idx_h, ones_h, zz_h, out_h, ix_v, lix_v, st_v, acc):
        c = lax.axis_index("c")
        s = lax.axis_index("s")
        wid = s * NC + c
        pltpu.sync_copy(ones_h, st_v)

        def window(w, _):
            base_node = w * WN
            pltpu.sync_copy(zz_h.at[pl.ds(s * 56, 56)],
                            acc.at[pl.ds(s * 56, 56)])

            @pl.when(s == NS - 1)
            def _():
                pltpu.sync_copy(zz_h.at[pl.ds(896, WR - 896)],
                                acc.at[pl.ds(896, WR - 896)])
            plsc.subcore_barrier()

            def chunk(k, _):
                base = wid * EPT + k * G
                pltpu.sync_copy(idx_h.at[pl.ds(base, G)], ix_v)

                def grp(t, _):
                    li = ix_v[pl.ds(t * LN, LN)] - base_node
                    ok = (li >= 0) & (li < WN)
                    li = jnp.where(ok, li, jnp.full((LN,), WN, i32))
                    j = t // (SUB // LN)
                    r = t - j * (SUB // LN)
                    lix_v[j, pl.ds(r * LN, LN)] = li
                    return 0
                lax.fori_loop(0, G // LN, grp, 0)
                for j in range(NSUB):
                    pltpu.sync_copy(st_v.at[pl.ds(j * SUB, SUB)],
                                    acc.at[lix_v.at[j]], add=True)
                return 0
            lax.fori_loop(0, EPT // G, chunk, 0)
            plsc.subcore_barrier()
            pltpu.sync_copy(acc.at[pl.ds(s * 56, 56)],
                            out_h.at[c, w, pl.ds(s * 56, 56)])

            @pl.when(s == NS - 1)
            def _():
                pltpu.sync_copy(acc.at[pl.ds(896, WR - 896)],
                                out_h.at[c, w, pl.ds(896, WR - 896)])
            plsc.subcore_barrier()
            return 0
        lax.fori_loop(0, NWIN, window, 0)

    return run(idx, ones_g, zeros_w)


def _agg(table, gidx, ab_idx, ltri, width, w=None):
    rows = sc_gather(table, gidx, width, E, 80)
    pref = tc_prefix(rows, ltri, w, width)
    pa, pb = sc_gather2(pref, pref, ab_idx[0], ab_idx[1], width, NB, 80)
    return pa, pb


def kernel(in_feat, edge_index, labels, train_mask, W1, b1, W2, b2):
    src = edge_index[0]
    dst = edge_index[1]

    perm = jnp.argsort(dst)
    dstS = dst[perm]
    srcS = src[perm]
    perm2 = jnp.argsort(src)
    srcS2 = src[perm2]
    dstS2 = dst[perm2]

    ltri = (jnp.arange(BP)[:, None] >= jnp.arange(BP)[None, :]).astype(f32)
    ones_g = jnp.ones((400, 16), f32)
    zeros_w = jnp.zeros((WR, 16), f32)
    w2p = jnp.pad(W2, ((0, 0), (0, CP - C)))
    w2pt = w2p.T
    b2p = jnp.pad(b2, (0, CP - C))
    labels2 = labels.reshape(N, 1)

    def pad_idx(a):
        return jnp.pad(a.reshape(N), (0, NB - N))

    histD = sc_win16(dstS, ones_g, zeros_w)
    histS = sc_win16(srcS2, ones_g, zeros_w)
    countsD, dinv0 = tc_winred(histD, 1.0, True)
    (countsS,) = tc_winred(histS, 1.0, False)
    aD, bD, amD, bmD = tc_ends(countsD.reshape(80, 125))
    aS, bS, amS, bmS = tc_ends(countsS.reshape(80, 125))
    abD = (pad_idx(aD), pad_idx(bD))
    abS = (pad_idx(aS), pad_idx(bS))
    amD, bmD = amD.reshape(N, 1), bmD.reshape(N, 1)
    amS, bmS = amS.reshape(N, 1), bmS.reshape(N, 1)

    p1 = tc_mm1(in_feat, W1)
    p1s = tc_scale(dinv0, p1)
    pa1, pb1 = _agg(p1s, srcS, abD, ltri, H)
    h1, p2p, p2s = tc_z1(pa1, pb1, amD, bmD, p1, dinv0, b1, w2p, True)
    pa2, pb2 = _agg(p2s, srcS, abD, ltri, 128)
    gp, gsp = tc_softmax(pa2, pb2, amD, bmD, p2p, dinv0, b2p, labels2)

    pa3, pb3 = _agg(gsp, dstS2, abS, ltri, 128)
    td, ts, dnsl = tc_dz1(pa3, pb3, amS, bmS, gp, p1, h1, dinv0, w2pt, p2p)
    ud, vs = sc_gather2(td, ts, dstS, srcS, TW, E, 80)
    t1rows = tc_dots(ud, vs)
    ud2, vs2 = sc_gather2(td, ts, dstS2, srcS2, TW, E, 80)
    t1rows2 = tc_dots(ud2, vs2)
    prefd = tc_prefix1(t1rows, ltri)
    prefs = tc_prefix1(t1rows2, ltri)
    pads_, pbds = sc_gather2(prefd, prefd, abD[0], abD[1], 128, NB, 80)
    pass_, pbss = sc_gather2(prefs, prefs, abS[0], abS[1], 128, NB, 80)
    ddegT = tc_ddeg(pass_, pbss, pads_, pbds, amS, bmS, amD, bmD,
                    dinv0, dnsl)

    dd = sc_gather(ddegT, dstS, 128, E, 80)
    ew, ewrows = tc_edgeup(t1rows, dd)

    prefe = tc_prefix1(ewrows, ltri)
    pae, pbe = sc_gather2(prefe, prefe, abD[0], abD[1], 128, NB, 80)
    dinv2, p1t2 = tc_deg2scale(pae, pbe, amD, bmD, p1)
    pa4, pb4 = _agg(p1t2, srcS, abD, ltri, H, w=ew)
    p2bp, p2bs = tc_z1(pa4, pb4, amD, bmD, p1, dinv2, b1, w2p, False)
    pa5, pb5 = _agg(p2bs, srcS, abD, ltri, 128, w=ew)
    return tc_final(pa5, pb5, amD, bmD, p2bp, dinv2, b2)
